# baseline (speedup 1.0000x reference)
import jax
import jax.numpy as jnp
from jax import lax
from jax.experimental import pallas as pl
from jax.experimental.pallas import tpu as pltpu
from jax.experimental.pallas import tpu_sc as plsc

_NUM_DETECTORS = 4096
_MAX_ROUNDS = 64
_DIM = 256
_B = 16
_SYN_LEN = 4096
_NRND = _MAX_ROUNDS + 1

_NC = 1
_NS = 8
_NW = _NC * _NS
_L = 16
_CHUNK = _SYN_LEN // _NW
_G = _CHUNK // _L
_W1 = _CHUNK + 8
_W1MAX = _SYN_LEN - _W1


def _dots_tc(det_ref, rnd_ref, proj_ref, alpha_ref, adet_ref, arnd_ref):
    a = alpha_ref[0, 0]
    proj = proj_ref[...]
    dn = (((1,), (1,)), ((), ()))
    adet = lax.dot_general(proj, det_ref[...], dn,
                           preferred_element_type=jnp.float32)
    arnd = lax.dot_general(proj, rnd_ref[...], dn,
                           preferred_element_type=jnp.float32)
    adet_ref[...] = a * adet
    arnd_ref[...] = a * arnd


def _sc_body(syn_hbm, mask_hbm, rlist_hbm, adet_hbm, arnd_hbm, out_hbm,
             adet_v, arnd_v, r_v, syn_v, mask_v, out_v, sem, sem_r):
    wid = lax.axis_index("s") * _NC + lax.axis_index("c")
    base = wid * _CHUNK
    spec_start = pl.multiple_of(jnp.minimum(base, _W1MAX), 8)

    copies = [
        pltpu.async_copy(adet_hbm.at[pl.ds(spec_start, _W1)],
                         adet_v.at[pl.ds(0, _W1)], sem),
        pltpu.async_copy(adet_hbm.at[pl.ds(0, _CHUNK)],
                         adet_v.at[pl.ds(_W1, _CHUNK)], sem),
        pltpu.async_copy(arnd_hbm, arnd_v, sem),
        pltpu.async_copy(syn_hbm.at[:, pl.ds(base, _CHUNK)], syn_v, sem),
        pltpu.async_copy(mask_hbm.at[:, pl.ds(base, _CHUNK)], mask_v, sem),
    ]
    pltpu.async_copy(rlist_hbm, r_v, sem_r).wait()

    r = r_v[...]
    d = lax.div(jnp.full((_L,), _SYN_LEN, jnp.int32), r)
    s_r = jnp.max(r)
    s_d = lax.div(jnp.int32(_SYN_LEN), s_r)
    start1a = pl.multiple_of(
        jnp.minimum(lax.rem(base, s_d) & jnp.int32(~7), jnp.int32(_W1MAX)), 8)

    for c in copies:
        c.wait()

    @pl.when(start1a != spec_start)
    def _refetch():
        pltpu.sync_copy(adet_hbm.at[pl.ds(start1a, _W1)],
                        adet_v.at[pl.ds(0, _W1)])

    for g in range(_G):
        sl = pl.ds(g * _L, _L)
        p = lax.broadcasted_iota(jnp.int32, (_L,), 0) + (base + g * _L)
        q = lax.div(p, d)
        det_id = p - q * d
        rnd_id = jnp.minimum(q + 1, _MAX_ROUNDS)
        det_local = jnp.where(det_id >= start1a, det_id - start1a,
                              det_id + _W1)
        pe = (plsc.load_gather(adet_v, [det_local]) +
              plsc.load_gather(arnd_v, [rnd_id]))
        for b in range(_B):
            out_v[b, sl] = syn_v[b, sl] + mask_v[b, sl] * pe

    pltpu.sync_copy(out_v, out_hbm.at[:, pl.ds(base, _CHUNK)])


@jax.jit
def kernel(syn_bits, r_list, mask, det_emb_w, rnd_emb_w, proj_w, alpha):
    alpha2d = jnp.reshape(alpha, (1, 1)).astype(jnp.float32)

    adet, arnd = pl.pallas_call(
        _dots_tc,
        out_shape=(
            jax.ShapeDtypeStruct((1, _NUM_DETECTORS), jnp.float32),
            jax.ShapeDtypeStruct((1, _NRND), jnp.float32),
        ),
    )(det_emb_w, rnd_emb_w, proj_w, alpha2d)
    adet = jnp.reshape(adet, (_NUM_DETECTORS,))
    arnd = jnp.reshape(arnd, (_NRND,))

    mesh = plsc.VectorSubcoreMesh(core_axis_name="c", subcore_axis_name="s",
                                  num_cores=_NC, num_subcores=_NS)
    sc = pl.kernel(
        _sc_body,
        out_type=jax.ShapeDtypeStruct((_B, _SYN_LEN), jnp.float32),
        mesh=mesh,
        compiler_params=pltpu.CompilerParams(needs_layout_passes=False),
        scratch_types=[
            pltpu.VMEM((_W1 + _CHUNK,), jnp.float32),
            pltpu.VMEM((_NRND,), jnp.float32),
            pltpu.VMEM((_L,), jnp.int32),
            pltpu.VMEM((_B, _CHUNK), jnp.float32),
            pltpu.VMEM((_B, _CHUNK), jnp.float32),
            pltpu.VMEM((_B, _CHUNK), jnp.float32),
            pltpu.SemaphoreType.DMA,
            pltpu.SemaphoreType.DMA,
        ],
    )
    return sc(syn_bits, mask, r_list, adet, arnd)


# speedup vs baseline: 1.1012x; 1.1012x over previous
import jax
import jax.numpy as jnp
from jax import lax
from jax.experimental import pallas as pl
from jax.experimental.pallas import tpu as pltpu
from jax.experimental.pallas import tpu_sc as plsc

_NUM_DETECTORS = 4096
_MAX_ROUNDS = 64
_DIM = 256
_B = 16
_SYN_LEN = 4096
_NRND = _MAX_ROUNDS + 1

_NC = 1
_NS = 16
_NW = _NC * _NS
_L = 16
_CHUNK = _SYN_LEN // _NW
_G = _CHUNK // _L
_W1 = _CHUNK + 8
_W1MAX = _SYN_LEN - _W1


def _dots_tc(det_ref, rnd_ref, proj_ref, alpha_ref, adet_ref, arnd_ref):
    a = alpha_ref[0, 0]
    proj = proj_ref[...]
    dn = (((1,), (1,)), ((), ()))
    adet = lax.dot_general(proj, det_ref[...], dn,
                           preferred_element_type=jnp.float32)
    arnd = lax.dot_general(proj, rnd_ref[...], dn,
                           preferred_element_type=jnp.float32)
    adet_ref[...] = a * adet
    arnd_ref[...] = a * arnd


def _sc_body(syn_hbm, mask_hbm, rlist_hbm, adet_hbm, arnd_hbm, out_hbm,
             adet_v, arnd_v, r_v, syn_v, mask_v, out_v, sem, sem_r):
    wid = lax.axis_index("s") * _NC + lax.axis_index("c")
    base = wid * _CHUNK
    spec_start = pl.multiple_of(jnp.minimum(base, _W1MAX), 8)

    copies = [
        pltpu.async_copy(adet_hbm.at[pl.ds(spec_start, _W1)],
                         adet_v.at[pl.ds(0, _W1)], sem),
        pltpu.async_copy(adet_hbm.at[pl.ds(0, _CHUNK)],
                         adet_v.at[pl.ds(_W1, _CHUNK)], sem),
        pltpu.async_copy(arnd_hbm, arnd_v, sem),
        pltpu.async_copy(syn_hbm.at[:, pl.ds(base, _CHUNK)], syn_v, sem),
        pltpu.async_copy(mask_hbm.at[:, pl.ds(base, _CHUNK)], mask_v, sem),
    ]
    pltpu.async_copy(rlist_hbm, r_v, sem_r).wait()

    r = r_v[...]
    d = lax.div(jnp.full((_L,), _SYN_LEN, jnp.int32), r)
    s_r = jnp.max(r)
    s_d = lax.div(jnp.int32(_SYN_LEN), s_r)
    start1a = pl.multiple_of(
        jnp.minimum(lax.rem(base, s_d) & jnp.int32(~7), jnp.int32(_W1MAX)), 8)

    for c in copies:
        c.wait()

    @pl.when(start1a != spec_start)
    def _refetch():
        pltpu.sync_copy(adet_hbm.at[pl.ds(start1a, _W1)],
                        adet_v.at[pl.ds(0, _W1)])

    for g in range(_G):
        sl = pl.ds(g * _L, _L)
        p = lax.broadcasted_iota(jnp.int32, (_L,), 0) + (base + g * _L)
        q = lax.div(p, d)
        det_id = p - q * d
        rnd_id = jnp.minimum(q + 1, _MAX_ROUNDS)
        det_local = jnp.where(det_id >= start1a, det_id - start1a,
                              det_id + _W1)
        pe = (plsc.load_gather(adet_v, [det_local]) +
              plsc.load_gather(arnd_v, [rnd_id]))
        for b in range(_B):
            out_v[b, sl] = syn_v[b, sl] + mask_v[b, sl] * pe

    pltpu.sync_copy(out_v, out_hbm.at[:, pl.ds(base, _CHUNK)])


@jax.jit
def kernel(syn_bits, r_list, mask, det_emb_w, rnd_emb_w, proj_w, alpha):
    alpha2d = jnp.reshape(alpha, (1, 1)).astype(jnp.float32)

    adet, arnd = pl.pallas_call(
        _dots_tc,
        out_shape=(
            jax.ShapeDtypeStruct((1, _NUM_DETECTORS), jnp.float32),
            jax.ShapeDtypeStruct((1, _NRND), jnp.float32),
        ),
    )(det_emb_w, rnd_emb_w, proj_w, alpha2d)
    adet = jnp.reshape(adet, (_NUM_DETECTORS,))
    arnd = jnp.reshape(arnd, (_NRND,))

    mesh = plsc.VectorSubcoreMesh(core_axis_name="c", subcore_axis_name="s",
                                  num_cores=_NC, num_subcores=_NS)
    sc = pl.kernel(
        _sc_body,
        out_type=jax.ShapeDtypeStruct((_B, _SYN_LEN), jnp.float32),
        mesh=mesh,
        compiler_params=pltpu.CompilerParams(needs_layout_passes=False),
        scratch_types=[
            pltpu.VMEM((_W1 + _CHUNK,), jnp.float32),
            pltpu.VMEM((_NRND,), jnp.float32),
            pltpu.VMEM((_L,), jnp.int32),
            pltpu.VMEM((_B, _CHUNK), jnp.float32),
            pltpu.VMEM((_B, _CHUNK), jnp.float32),
            pltpu.VMEM((_B, _CHUNK), jnp.float32),
            pltpu.SemaphoreType.DMA,
            pltpu.SemaphoreType.DMA,
        ],
    )
    return sc(syn_bits, mask, r_list, adet, arnd)


# speedup vs baseline: 1.1116x; 1.0094x over previous
import jax
import jax.numpy as jnp
from jax import lax
from jax.experimental import pallas as pl
from jax.experimental.pallas import tpu as pltpu
from jax.experimental.pallas import tpu_sc as plsc

_NUM_DETECTORS = 4096
_MAX_ROUNDS = 64
_DIM = 256
_B = 16
_SYN_LEN = 4096
_NRND = _MAX_ROUNDS + 1

_NC = 1
_NS = 16
_NW = _NC * _NS
_L = 16
_CHUNK = _SYN_LEN // _NW
_G = _CHUNK // _L
_W1 = _CHUNK + 8
_W1MAX = _SYN_LEN - _W1


def _dots_tc(det_ref, rnd_ref, proj_ref, alpha_ref, adet_ref, arnd_ref):
    a = alpha_ref[0, 0]
    proj = proj_ref[...]
    dn = (((1,), (1,)), ((), ()))
    adet = lax.dot_general(proj, det_ref[...], dn,
                           preferred_element_type=jnp.float32)
    arnd = lax.dot_general(proj, rnd_ref[...], dn,
                           preferred_element_type=jnp.float32)
    adet_ref[...] = a * adet
    arnd_ref[...] = a * arnd


def _sc_body(syn_hbm, mask_hbm, rlist_hbm, adet_hbm, arnd_hbm, out_hbm,
             adet_v, arnd_v, r_v, syn_v, mask_v, out_v, sem, sem_r):
    wid = lax.axis_index("s") * _NC + lax.axis_index("c")
    base = wid * _CHUNK
    spec_start = pl.multiple_of(jnp.minimum(base, _W1MAX), 8)

    copies = [
        pltpu.async_copy(adet_hbm.at[pl.ds(spec_start, _W1)],
                         adet_v.at[pl.ds(0, _W1)], sem),
        pltpu.async_copy(adet_hbm.at[pl.ds(0, _CHUNK)],
                         adet_v.at[pl.ds(_W1, _CHUNK)], sem),
        pltpu.async_copy(arnd_hbm, arnd_v, sem),
        pltpu.async_copy(syn_hbm.at[:, pl.ds(base, _CHUNK)], syn_v, sem),
        pltpu.async_copy(mask_hbm.at[:, pl.ds(base, _CHUNK)], mask_v, sem),
    ]
    pltpu.async_copy(rlist_hbm, r_v, sem_r).wait()

    r = r_v[...]
    d = lax.div(jnp.full((_L,), _SYN_LEN, jnp.int32), r)
    s_r = jnp.max(r)
    s_d = lax.div(jnp.int32(_SYN_LEN), s_r)
    start1a = pl.multiple_of(
        jnp.minimum(lax.rem(base, s_d) & jnp.int32(~7), jnp.int32(_W1MAX)), 8)

    for c in copies:
        c.wait()

    @pl.when(start1a != spec_start)
    def _refetch():
        pltpu.sync_copy(adet_hbm.at[pl.ds(start1a, _W1)],
                        adet_v.at[pl.ds(0, _W1)])

    half = _CHUNK // 2
    for g in range(_G):
        sl = pl.ds(g * _L, _L)
        p = lax.broadcasted_iota(jnp.int32, (_L,), 0) + (base + g * _L)
        q = lax.div(p, d)
        det_id = p - q * d
        rnd_id = jnp.minimum(q + 1, _MAX_ROUNDS)
        det_local = jnp.where(det_id >= start1a, det_id - start1a,
                              det_id + _W1)
        pe = (plsc.load_gather(adet_v, [det_local]) +
              plsc.load_gather(arnd_v, [rnd_id]))
        for b in range(_B):
            out_v[b, sl] = syn_v[b, sl] + mask_v[b, sl] * pe
        if g == _G // 2 - 1:
            out1 = pltpu.async_copy(out_v.at[:, pl.ds(0, half)],
                                    out_hbm.at[:, pl.ds(base, half)], sem_r)

    out2 = pltpu.async_copy(out_v.at[:, pl.ds(half, half)],
                            out_hbm.at[:, pl.ds(base + half, half)], sem_r)
    out1.wait()
    out2.wait()


@jax.jit
def kernel(syn_bits, r_list, mask, det_emb_w, rnd_emb_w, proj_w, alpha):
    alpha2d = jnp.reshape(alpha, (1, 1)).astype(jnp.float32)

    adet, arnd = pl.pallas_call(
        _dots_tc,
        out_shape=(
            jax.ShapeDtypeStruct((1, _NUM_DETECTORS), jnp.float32),
            jax.ShapeDtypeStruct((1, _NRND), jnp.float32),
        ),
    )(det_emb_w, rnd_emb_w, proj_w, alpha2d)
    adet = jnp.reshape(adet, (_NUM_DETECTORS,))
    arnd = jnp.reshape(arnd, (_NRND,))

    mesh = plsc.VectorSubcoreMesh(core_axis_name="c", subcore_axis_name="s",
                                  num_cores=_NC, num_subcores=_NS)
    sc = pl.kernel(
        _sc_body,
        out_type=jax.ShapeDtypeStruct((_B, _SYN_LEN), jnp.float32),
        mesh=mesh,
        compiler_params=pltpu.CompilerParams(needs_layout_passes=False),
        scratch_types=[
            pltpu.VMEM((_W1 + _CHUNK,), jnp.float32),
            pltpu.VMEM((_NRND,), jnp.float32),
            pltpu.VMEM((_L,), jnp.int32),
            pltpu.VMEM((_B, _CHUNK), jnp.float32),
            pltpu.VMEM((_B, _CHUNK), jnp.float32),
            pltpu.VMEM((_B, _CHUNK), jnp.float32),
            pltpu.SemaphoreType.DMA,
            pltpu.SemaphoreType.DMA,
        ],
    )
    return sc(syn_bits, mask, r_list, adet, arnd)
